# Initial kernel scaffold; baseline (speedup 1.0000x reference)
import jax
import jax.numpy as jnp
from jax.experimental import pallas as pl


def kernel(ids, feats, adj, att_w1, fcx_w1, fcn_w1, att_w2, fcx_w2, fcn_w2, fc_w, fc_b):
    raise NotImplementedError("write your pallas kernel here")


# speedup vs baseline: 3.6769x; 3.6769x over previous
import functools

import numpy as np
import jax
import jax.numpy as jnp
from jax import lax
from jax.experimental import pallas as pl
from jax.experimental.pallas import tpu as pltpu
from jax.experimental.pallas import tpu_sc as plsc

_N_NODES = 100000
_DEG = 32
_D = 128
_H = 32
_SEEDS = 1024
_K1 = 25
_K2 = 10
_B1 = _SEEDS * _K1
_B2 = _B1 * _K2

_NCORE = 2
_NSUB = 16
_NW = _NCORE * _NSUB

def _sample_cols(pad):
    key = jax.random.key(42)
    p0 = jax.random.permutation(jax.random.fold_in(key, 0), _DEG)[:_K1].astype(jnp.int32)
    p1 = jax.random.permutation(jax.random.fold_in(key, 1), _DEG)[:_K2].astype(jnp.int32)
    c1 = jnp.zeros((32,), jnp.int32).at[:_K1].set(p0)
    c2 = jnp.zeros((16,), jnp.int32).at[:_K2].set(p1)
    return c1, c2

def _sc_mesh():
    return plsc.VectorSubcoreMesh(core_axis_name="c", subcore_axis_name="s")


def _wid():
    return lax.axis_index("s") * _NCORE + lax.axis_index("c")


def _expand1_body(ids_hbm, adjflat_hbm, cols_hbm, out_hbm, ids_v, cols_v, idx_v, val_v, sem):
    n = _SEEDS // _NW
    base = _wid() * n
    pltpu.sync_copy(ids_hbm.at[pl.ds(base, n)], ids_v)
    pltpu.sync_copy(cols_hbm, cols_v)
    cv = [cols_v[pl.ds(0, 16)], cols_v[pl.ds(16, 16)]]
    for k in range(_K1):
        c = cv[k // 16][k % 16]
        for j in range(n // 16):
            idx_v[pl.ds(k * n + j * 16, 16)] = ids_v[pl.ds(j * 16, 16)] * _DEG + c
    handles = []
    for k in range(_K1):
        handles.append(pltpu.async_copy(
            adjflat_hbm.at[idx_v.at[pl.ds(k * n, n)]], val_v.at[pl.ds(k * n, n)], sem))
    for h in handles:
        h.wait()
    for k in range(_K1):
        pltpu.sync_copy(val_v.at[pl.ds(k * n, n)], out_hbm.at[pl.ds(k * _SEEDS + base, n)])


def _expand1(ids, adjflat, cols1):
    n = _SEEDS // _NW
    f = pl.kernel(
        _expand1_body,
        out_type=jax.ShapeDtypeStruct((_B1,), jnp.int32),
        mesh=_sc_mesh(),
        compiler_params=pltpu.CompilerParams(needs_layout_passes=False, use_tc_tiling_on_sc=False),
        scratch_types=[
            pltpu.VMEM((n,), jnp.int32),
            pltpu.VMEM((32,), jnp.int32),
            pltpu.VMEM((_K1 * n,), jnp.int32),
            pltpu.VMEM((_K1 * n,), jnp.int32),
            pltpu.SemaphoreType.DMA,
        ],
    )
    return f(ids, adjflat, cols1)


def _expand2_body(ids_hbm, adjflat_hbm, cols_hbm, out_hbm, ids_v, cols_v, idx_v, val_v, sem):
    n = _B1 // _NW
    ne = n * _K2
    nrow = ne // 80
    base = _wid() * n
    pltpu.sync_copy(ids_hbm.at[pl.ds(base, n)], ids_v)
    pltpu.sync_copy(cols_hbm, cols_v.at[pl.ds(0, 16)])
    lanes = lax.iota(jnp.int32, 16)
    ckv = []
    for p in range(5):
        kpat = (lanes + p * 16) % _K2
        ckv.append(plsc.load_gather(cols_v, [kpat]))
    for g in range(ne // 16):
        jv = lanes + g * 16
        iv = jv // _K2
        idv = plsc.load_gather(ids_v, [iv])
        idx_v[pl.ds(g * 16, 16)] = idv * _DEG + ckv[g % 5]
    handles = []
    for r in range(nrow):
        handles.append(pltpu.async_copy(
            adjflat_hbm.at[idx_v.at[pl.ds(r * 80, 80)]], val_v.at[pl.ds(r * 80, 80)], sem))
    for h in handles:
        h.wait()
    pltpu.sync_copy(val_v, out_hbm.at[pl.ds(_wid() * ne, ne)])


def _expand2(ids1, adjflat, cols2):
    n = _B1 // _NW
    nrow = n * _K2 // 80
    f = pl.kernel(
        _expand2_body,
        out_type=jax.ShapeDtypeStruct((_B2,), jnp.int32),
        mesh=_sc_mesh(),
        compiler_params=pltpu.CompilerParams(needs_layout_passes=False, use_tc_tiling_on_sc=False),
        scratch_types=[
            pltpu.VMEM((n,), jnp.int32),
            pltpu.VMEM((128,), jnp.int32),
            pltpu.VMEM((nrow * 80,), jnp.int32),
            pltpu.VMEM((nrow * 80,), jnp.int32),
            pltpu.SemaphoreType.DMA,
        ],
    )
    return f(ids1, adjflat, cols2)


def _gather_fg_body(n, feats_hbm, a_hbm, idx_hbm, f_out, g_out, idx_v, f_v, g_v, sem):
    ch = min(80, n)
    nch = n // ch
    base = _wid() * n
    pltpu.sync_copy(idx_hbm.at[pl.ds(base, n)], idx_v)
    handles = []
    for c in range(nch):
        ix = idx_v.at[pl.ds(c * ch, ch)]
        handles.append(pltpu.async_copy(feats_hbm.at[ix], f_v.at[pl.ds(c * ch, ch)], sem))
        handles.append(pltpu.async_copy(a_hbm.at[ix], g_v.at[pl.ds(c * ch, ch)], sem))
    for h in handles:
        h.wait()
    pltpu.sync_copy(f_v, f_out.at[pl.ds(base, n), :])
    pltpu.sync_copy(g_v, g_out.at[pl.ds(base, n), :])


def _gather_fg(feats, a, idx, b):
    n = b // _NW
    ch = min(80, n)
    f = pl.kernel(
        functools.partial(_gather_fg_body, n),
        out_type=(jax.ShapeDtypeStruct((b, _D), jnp.float32),
                  jax.ShapeDtypeStruct((b, _H), jnp.float32)),
        mesh=_sc_mesh(),
        compiler_params=pltpu.CompilerParams(needs_layout_passes=False, use_tc_tiling_on_sc=False),
        scratch_types=[
            pltpu.VMEM((n,), jnp.int32),
            pltpu.VMEM((n, _D), jnp.float32),
            pltpu.VMEM((n, _H), jnp.float32),
            pltpu.SemaphoreType.DMA,
        ],
    )
    return f(feats, a, idx)


def _gather_g_body(a_hbm, idx_hbm, g_out, idx_v, g_v, sem):
    n = _B2 // _NW
    sup = 2000
    ch = 80
    base = _wid() * n
    pltpu.sync_copy(idx_hbm.at[pl.ds(base, n)], idx_v)
    for s in range(n // sup):
        handles = []
        for c in range(sup // ch):
            ix = idx_v.at[pl.ds(s * sup + c * ch, ch)]
            handles.append(pltpu.async_copy(a_hbm.at[ix], g_v.at[pl.ds(c * ch, ch)], sem))
        for h in handles:
            h.wait()
        pltpu.sync_copy(g_v, g_out.at[pl.ds(base + s * sup, sup), :])


def _gather_g(a, idx):
    n = _B2 // _NW
    f = pl.kernel(
        _gather_g_body,
        out_type=jax.ShapeDtypeStruct((_B2, _H), jnp.float32),
        mesh=_sc_mesh(),
        compiler_params=pltpu.CompilerParams(needs_layout_passes=False, use_tc_tiling_on_sc=False),
        scratch_types=[
            pltpu.VMEM((n,), jnp.int32),
            pltpu.VMEM((2000, _H), jnp.float32),
            pltpu.SemaphoreType.DMA,
        ],
    )
    return f(a, idx)


def _wreduce_body(feats_hbm, idx_hbm, ws_hbm, out_hbm, idx_v, ws_v, rows_v, acc_v, sem):
    n = _B1 // _NW
    P = 40
    base = _wid() * n
    pltpu.sync_copy(idx_hbm.at[pl.ds(base * _K2, n * _K2)], idx_v)
    pltpu.sync_copy(ws_hbm.at[pl.ds(base, n), :], ws_v)
    for s in range(n // P):
        handles = []
        for c in range(P * _K2 // 80):
            ix = idx_v.at[pl.ds((s * (P * _K2 // 80) + c) * 80, 80)]
            handles.append(pltpu.async_copy(feats_hbm.at[ix], rows_v.at[pl.ds(c * 80, 80)], sem))
        for h in handles:
            h.wait()

        def body(i, _):
            wsrow = ws_v[s * P + i, pl.ds(0, 16)]
            acc = [jnp.zeros((16,), jnp.float32) for _ in range(_D // 16)]
            for k in range(_K2):
                w = wsrow[k]
                for v in range(_D // 16):
                    acc[v] = acc[v] + w * rows_v[i * _K2 + k, pl.ds(v * 16, 16)]
            for v in range(_D // 16):
                acc_v[i, pl.ds(v * 16, 16)] = acc[v]
            return 0

        lax.fori_loop(0, P, body, 0)
        pltpu.sync_copy(acc_v, out_hbm.at[pl.ds(base + s * P, P), :])


def _wreduce(feats, ids2, ws):
    n = _B1 // _NW
    P = 40
    f = pl.kernel(
        _wreduce_body,
        out_type=jax.ShapeDtypeStruct((_B1, _D), jnp.float32),
        mesh=_sc_mesh(),
        compiler_params=pltpu.CompilerParams(needs_layout_passes=False, use_tc_tiling_on_sc=False),
        scratch_types=[
            pltpu.VMEM((n * _K2,), jnp.int32),
            pltpu.VMEM((n, 16), jnp.float32),
            pltpu.VMEM((P * _K2, _D), jnp.float32),
            pltpu.VMEM((P, _D), jnp.float32),
            pltpu.SemaphoreType.DMA,
        ],
    )
    return f(feats, ids2, ws)


def _amm_body(f_ref, w_ref, o_ref):
    o_ref[...] = jnp.dot(f_ref[...], w_ref[...], preferred_element_type=jnp.float32)


def _amm(feats, att_w1):
    blk = 2000
    return pl.pallas_call(
        _amm_body,
        grid=(_N_NODES // blk,),
        in_specs=[pl.BlockSpec((blk, _D), lambda i: (i, 0)),
                  pl.BlockSpec((_D, _H), lambda i: (0, 0))],
        out_specs=pl.BlockSpec((blk, _H), lambda i: (i, 0)),
        out_shape=jax.ShapeDtypeStruct((_N_NODES, _H), jnp.float32),
    )(feats, att_w1)


def _ws_body(g2_ref, g1_ref, o_ref):
    g1 = g1_ref[...]
    s = [jnp.sum(g2_ref[:, _H * k:_H * (k + 1)] * g1, axis=1, keepdims=True)
         for k in range(_K2)]
    m = s[0]
    for k in range(1, _K2):
        m = jnp.maximum(m, s[k])
    e = [jnp.exp(sk - m) for sk in s]
    den = e[0]
    for k in range(1, _K2):
        den = den + e[k]
    inv = 1.0 / den
    cols = lax.broadcasted_iota(jnp.int32, o_ref.shape, 1)
    out = jnp.zeros(o_ref.shape, jnp.float32)
    for k in range(_K2):
        out = out + jnp.where(cols == k, e[k] * inv, 0.0)
    o_ref[...] = out


def _ws(g2flat, g1):
    blk = 3200
    return pl.pallas_call(
        _ws_body,
        grid=(_B1 // blk,),
        in_specs=[pl.BlockSpec((blk, _K2 * _H), lambda i: (i, 0)),
                  pl.BlockSpec((blk, _H), lambda i: (i, 0))],
        out_specs=pl.BlockSpec((blk, 16), lambda i: (i, 0)),
        out_shape=jax.ShapeDtypeStruct((_B1, 16), jnp.float32),
    )(g2flat, g1)


def _tail_body(f0_ref, g0_ref, f1_ref, g1r_ref, ag1_ref,
               axw1_ref, anw1_ref, aw2_ref, axw2_ref, anw2_ref,
               fcw_ref, fcb_ref, o_ref):
    F0 = f0_ref[...]
    G0 = g0_ref[...]
    axw1 = axw1_ref[...]
    anw1 = anw1_ref[...]

    s = [jnp.sum(g1r_ref[k] * G0, axis=1, keepdims=True) for k in range(_K1)]
    m = s[0]
    for k in range(1, _K1):
        m = jnp.maximum(m, s[k])
    e = [jnp.exp(sk - m) for sk in s]
    den = e[0]
    for k in range(1, _K1):
        den = den + e[k]
    inv = 1.0 / den
    agg0 = jnp.zeros(F0.shape, jnp.float32)
    for k in range(_K1):
        agg0 = agg0 + (e[k] * inv) * f1_ref[k]

    g0a = jnp.maximum(F0 @ axw1, 0.0)
    g0b = jnp.maximum(agg0 @ anw1, 0.0)

    aw2a = aw2_ref[0:_D, :]
    aw2b = aw2_ref[_D:2 * _D, :]
    x2 = g0a @ aw2a + g0b @ aw2b

    g1a, g1b, s2 = [], [], []
    for k in range(_K1):
        a = jnp.maximum(f1_ref[k] @ axw1, 0.0)
        b = jnp.maximum(ag1_ref[k] @ anw1, 0.0)
        n2 = a @ aw2a + b @ aw2b
        g1a.append(a)
        g1b.append(b)
        s2.append(jnp.sum(n2 * x2, axis=1, keepdims=True))
    m2 = s2[0]
    for k in range(1, _K1):
        m2 = jnp.maximum(m2, s2[k])
    e2 = [jnp.exp(sk - m2) for sk in s2]
    den2 = e2[0]
    for k in range(1, _K1):
        den2 = den2 + e2[k]
    inv2 = 1.0 / den2
    agg2a = jnp.zeros(F0.shape, jnp.float32)
    agg2b = jnp.zeros(F0.shape, jnp.float32)
    for k in range(_K1):
        w = e2[k] * inv2
        agg2a = agg2a + w * g1a[k]
        agg2b = agg2b + w * g1b[k]

    h0a = jnp.maximum(g0a @ axw2_ref[0:_D, :] + g0b @ axw2_ref[_D:2 * _D, :], 0.0)
    h0b = jnp.maximum(agg2a @ anw2_ref[0:_D, :] + agg2b @ anw2_ref[_D:2 * _D, :], 0.0)
    nrm = jnp.sqrt(jnp.sum(h0a * h0a, axis=1, keepdims=True)
                   + jnp.sum(h0b * h0b, axis=1, keepdims=True))
    sc = 1.0 / jnp.maximum(nrm, 1e-12)
    o_ref[...] = (h0a * sc) @ fcw_ref[0:_D, :] + (h0b * sc) @ fcw_ref[_D:2 * _D, :] + fcb_ref[...]


def _tail(F0, G0, F1t, G1t, AG1t, fcx_w1, fcn_w1, att_w2, fcx_w2, fcn_w2, fc_w, fc_b):
    S = 256
    nc = fc_w.shape[1]
    return pl.pallas_call(
        _tail_body,
        grid=(_SEEDS // S,),
        in_specs=[
            pl.BlockSpec((S, _D), lambda i: (i, 0)),
            pl.BlockSpec((S, _H), lambda i: (i, 0)),
            pl.BlockSpec((_K1, S, _D), lambda i: (0, i, 0)),
            pl.BlockSpec((_K1, S, _H), lambda i: (0, i, 0)),
            pl.BlockSpec((_K1, S, _D), lambda i: (0, i, 0)),
            pl.BlockSpec((_D, _D), lambda i: (0, 0)),
            pl.BlockSpec((_D, _D), lambda i: (0, 0)),
            pl.BlockSpec((2 * _D, _H), lambda i: (0, 0)),
            pl.BlockSpec((2 * _D, _D), lambda i: (0, 0)),
            pl.BlockSpec((2 * _D, _D), lambda i: (0, 0)),
            pl.BlockSpec((2 * _D, nc), lambda i: (0, 0)),
            pl.BlockSpec((1, nc), lambda i: (0, 0)),
        ],
        out_specs=pl.BlockSpec((S, nc), lambda i: (i, 0)),
        out_shape=jax.ShapeDtypeStruct((_SEEDS, nc), jnp.float32),
    )(F0, G0, F1t, G1t, AG1t, fcx_w1, fcn_w1, att_w2, fcx_w2, fcn_w2, fc_w, fc_b)


def kernel(ids, feats, adj, att_w1, fcx_w1, fcn_w1, att_w2, fcx_w2, fcn_w2, fc_w, fc_b):
    ids = ids.astype(jnp.int32)
    adjflat = adj.astype(jnp.int32).reshape(-1)
    cols1, cols2 = _sample_cols(None)

    ids1 = _expand1(ids, adjflat, cols1)
    ids2 = _expand2(ids1, adjflat, cols2)
    A = _amm(feats, att_w1)
    F0, G0 = _gather_fg(feats, A, ids, _SEEDS)
    F1, G1 = _gather_fg(feats, A, ids1, _B1)
    G2 = _gather_g(A, ids2)
    ws = _ws(G2.reshape(_B1, _K2 * _H), G1)
    AG1 = _wreduce(feats, ids2, ws)

    return _tail(F0, G0,
                 F1.reshape(_K1, _SEEDS, _D),
                 G1.reshape(_K1, _SEEDS, _H),
                 AG1.reshape(_K1, _SEEDS, _D),
                 fcx_w1, fcn_w1, att_w2, fcx_w2, fcn_w2, fc_w,
                 fc_b.reshape(1, -1))
